# baseline (speedup 1.0000x reference)
import functools
import jax
import jax.numpy as jnp
from jax import lax
from jax.experimental import pallas as pl
from jax.experimental.pallas import tpu as pltpu
from jax.experimental.pallas import tpu_sc as plsc

D_H = 128
CHUNK = 128
NC, NS = 2, 16
NW = NC * NS
LANES = 16

SPLIT_FN = lambda total: (0, total)


def _make_sc_segment_sum(n_nodes, e_pad, cpt0, cpt1):
  assert NS * (cpt0 + cpt1) * CHUNK == e_pad
  rows_per_tile = ((n_nodes + NS - 1) // NS + 7) // 8 * 8
  n_pad = rows_per_tile * NS
  full_zero_chunks = rows_per_tile // CHUNK
  rem_zero = rows_per_tile % CHUNK
  mesh = plsc.VectorSubcoreMesh(core_axis_name="c", subcore_axis_name="s",
                                num_cores=NC, num_subcores=NS)

  bufc = min(40, max(cpt0, cpt1))
  edges_per_tile = bufc * CHUNK

  @functools.partial(
      pl.kernel,
      out_type=[jax.ShapeDtypeStruct((n_pad, D_H), jnp.float32)] * NC,
      mesh=mesh,
      scratch_types=[
          pltpu.VMEM((max(edges_per_tile, 1),), jnp.int32),
          pltpu.VMEM((max(bufc, 1), 1, CHUNK), jnp.int32),
          pltpu.VMEM((max(edges_per_tile, 1),), jnp.float32),
          pltpu.VMEM((CHUNK, D_H), jnp.float32),
          pltpu.VMEM((CHUNK, D_H), jnp.float32),
          pltpu.VMEM_SHARED((n_pad, D_H), jnp.float32),
          pltpu.SemaphoreType.DMA,
          pltpu.SemaphoreType.DMA,
          pltpu.SemaphoreType.DMA,
          pltpu.SemaphoreType.DMA,
          pltpu.SemaphoreType.DMA,
      ],
  )
  def sc_kernel(y_hbm, src_hbm, dst_hbm, ew_hbm, out0_hbm, out1_hbm,
                src_v, dst_v, ew_v, rows_a, rows_b, agg_sh,
                gs_a, gs_b, ss_a, ss_b, is_sem):
    c = lax.axis_index("c")
    s = lax.axis_index("s")
    row0 = s * rows_per_tile

    zero16 = jnp.zeros((LANES,), jnp.float32)

    def zrow(i, carry):
      for f in range(D_H // LANES):
        rows_a[i, pl.ds(f * LANES, LANES)] = zero16
      return carry

    lax.fori_loop(0, CHUNK, zrow, 0)
    for k in range(full_zero_chunks):
      pltpu.sync_copy(rows_a, agg_sh.at[pl.ds(row0 + k * CHUNK, CHUNK)])
    if rem_zero:
      pltpu.sync_copy(rows_a.at[pl.ds(0, rem_zero)],
                      agg_sh.at[pl.ds(row0 + full_zero_chunks * CHUNK, rem_zero)])

    def issue_gather(ci, buf, gsem):
      return pltpu.async_copy(
          y_hbm.at[src_v.at[pl.ds(ci * CHUNK, CHUNK)]], buf, gsem)

    def issue_scatter(ci, buf, ssem):
      return pltpu.async_copy(buf, agg_sh.at[dst_v.at[ci, 0]], ssem, add=True)

    def do_scale(ci, buf):
      def scale(g, inner):
        wvec = ew_v[pl.ds(ci * CHUNK + g * LANES, LANES)]
        for j in range(LANES):
          wv = jnp.full((LANES,), wvec[j], jnp.float32)
          e = g * LANES + j
          for f in range(D_H // LANES):
            sl = pl.ds(f * LANES, LANES)
            buf[e, sl] = buf[e, sl] * wv
        return inner

      lax.fori_loop(0, CHUNK // LANES, scale, 0)

    def run(nchunks, tile_chunk0):
      assert nchunks % 2 == 0
      npairs = nchunks // 2
      nedges = nchunks * CHUNK
      ebase = tile_chunk0 * CHUNK
      ia = pltpu.async_copy(src_hbm.at[pl.ds(ebase, nedges)],
                            src_v.at[pl.ds(0, nedges)], is_sem)
      ib = pltpu.async_copy(dst_hbm.at[pl.ds(tile_chunk0, nchunks)],
                            dst_v.at[pl.ds(0, nchunks)], is_sem)
      ic = pltpu.async_copy(ew_hbm.at[pl.ds(ebase, nedges)],
                            ew_v.at[pl.ds(0, nedges)], is_sem)
      ia.wait()
      ib.wait()
      ic.wait()
      plsc.subcore_barrier()

      issue_gather(0, rows_a, gs_a)
      issue_gather(1, rows_b, gs_b)

      def pair_body(p, carry):
        ci = 2 * p
        pltpu.make_async_copy(y_hbm.at[src_v.at[pl.ds(0, CHUNK)]], rows_a,
                              gs_a).wait()
        do_scale(ci, rows_a)
        issue_scatter(ci, rows_a, ss_a)
        pltpu.make_async_copy(y_hbm.at[src_v.at[pl.ds(0, CHUNK)]], rows_b,
                              gs_b).wait()
        do_scale(ci + 1, rows_b)
        issue_scatter(ci + 1, rows_b, ss_b)

        @pl.when(p < npairs - 1)
        def _():
          pltpu.make_async_copy(rows_a, agg_sh.at[dst_v.at[0, 0]], ss_a).wait()
          issue_gather(ci + 2, rows_a, gs_a)
          pltpu.make_async_copy(rows_b, agg_sh.at[dst_v.at[0, 0]], ss_b).wait()
          issue_gather(ci + 3, rows_b, gs_b)

        return carry

      lax.fori_loop(0, npairs, pair_body, 0)
      pltpu.make_async_copy(rows_a, agg_sh.at[dst_v.at[0, 0]], ss_a).wait()
      pltpu.make_async_copy(rows_b, agg_sh.at[dst_v.at[0, 0]], ss_b).wait()

    def run_core(cpt, core_base):
      done = 0
      while done < cpt:
        step = min(bufc, cpt - done)
        run(step, core_base + s * cpt + done)
        done += step

    if cpt0 == cpt1:
      run_core(cpt0, c * NS * cpt0)
    else:
      if cpt0:
        @pl.when(c == 0)
        def _():
          run_core(cpt0, 0)
      if cpt1:
        @pl.when(c == 1)
        def _():
          run_core(cpt1, NS * cpt0)
    plsc.subcore_barrier()

    @pl.when(c == 0)
    def _():
      pltpu.sync_copy(agg_sh.at[pl.ds(row0, rows_per_tile)],
                      out0_hbm.at[pl.ds(row0, rows_per_tile)])

    @pl.when(c == 1)
    def _():
      pltpu.sync_copy(agg_sh.at[pl.ds(row0, rows_per_tile)],
                      out1_hbm.at[pl.ds(row0, rows_per_tile)])

  return sc_kernel


def _mm_body(x_ref, w_ref, y_ref, r_ref):
  acc = jnp.dot(x_ref[...], w_ref[...], preferred_element_type=jnp.float32)
  y_ref[...] = acc[:, :D_H]
  r_ref[...] = acc[:, D_H:]


def _head_body(agg0, agg1, r, x1, brel, w1t, b1, w4t, b4, w2et, w2x, b2, w3t, b3,
               out_ref, emb_ref):
  h = jnp.maximum(agg0[...] + agg1[...] + r[...] + brel[...], 0.0)
  h1 = jnp.maximum(
      jnp.dot(h, w1t[...], preferred_element_type=jnp.float32) + b1[...], 0.0)
  emb = jax.nn.sigmoid(
      jnp.dot(h1, w4t[...], preferred_element_type=jnp.float32) + b4[...])
  emb_ref[...] = emb
  h2 = jax.nn.sigmoid(
      jnp.dot(emb, w2et[...], preferred_element_type=jnp.float32)
      + x1[...] * w2x[...] + b2[...])
  out_ref[...] = jnp.maximum(
      jnp.dot(h2, w3t[...], preferred_element_type=jnp.float32) + b3[...], 0.0)


def kernel(x, x1, edge_weight, W_rel, b_rel, W_root, W1, b1, W4, b4, W2, b2, W3,
           b3, edge_index, loc_num):
  n, d = x.shape
  e = edge_index.shape[1]
  blk = 1000
  nblk = n // blk

  wcat = jnp.concatenate([W_rel, W_root], axis=0).T
  y, r = pl.pallas_call(
      _mm_body,
      grid=(nblk,),
      in_specs=[
          pl.BlockSpec((blk, d), lambda i: (i, 0)),
          pl.BlockSpec((d, 2 * D_H), lambda i: (0, 0)),
      ],
      out_specs=[
          pl.BlockSpec((blk, D_H), lambda i: (i, 0)),
          pl.BlockSpec((blk, D_H), lambda i: (i, 0)),
      ],
      out_shape=[
          jax.ShapeDtypeStruct((n, D_H), jnp.float32),
          jax.ShapeDtypeStruct((n, D_H), jnp.float32),
      ],
  )(x, wcat)

  total_cpt = ((e + NS * CHUNK - 1) // (NS * CHUNK) + 1) // 2 * 2
  cpt0, cpt1 = SPLIT_FN(total_cpt)
  e_pad = NS * (cpt0 + cpt1) * CHUNK
  pad = e_pad - e
  src = edge_index[0]
  dst = edge_index[1]
  ew = edge_weight
  if pad:
    zi = jnp.zeros((pad,), jnp.int32)
    src = jnp.concatenate([src, zi])
    dst = jnp.concatenate([dst, zi])
    ew = jnp.concatenate([ew, jnp.zeros((pad,), jnp.float32)])
  dst3 = dst.reshape(e_pad // CHUNK, 1, CHUNK)
  agg0, agg1 = _make_sc_segment_sum(n, e_pad, cpt0, cpt1)(y, src, dst3, ew)

  delta = (jnp.asarray(loc_num) - n).astype(jnp.float32)
  b_eff = (b_rel + delta).reshape(1, D_H)
  x1c = x1.reshape(n, 1)
  w1t = W1.T
  b1r = b1.reshape(1, 32)
  w4t = W4.T
  b4r = b4.reshape(1, 8)
  w2et = W2[:, :8].T
  w2x = W2[:, 8].reshape(1, 4)
  b2r = b2.reshape(1, 4)
  w3t = W3.T
  b3r = b3.reshape(1, 1)

  out, emb = pl.pallas_call(
      _head_body,
      grid=(nblk,),
      in_specs=[
          pl.BlockSpec((blk, D_H), lambda i: (i, 0)),
          pl.BlockSpec((blk, D_H), lambda i: (i, 0)),
          pl.BlockSpec((blk, D_H), lambda i: (i, 0)),
          pl.BlockSpec((blk, 1), lambda i: (i, 0)),
          pl.BlockSpec((1, D_H), lambda i: (0, 0)),
          pl.BlockSpec((D_H, 32), lambda i: (0, 0)),
          pl.BlockSpec((1, 32), lambda i: (0, 0)),
          pl.BlockSpec((32, 8), lambda i: (0, 0)),
          pl.BlockSpec((1, 8), lambda i: (0, 0)),
          pl.BlockSpec((8, 4), lambda i: (0, 0)),
          pl.BlockSpec((1, 4), lambda i: (0, 0)),
          pl.BlockSpec((1, 4), lambda i: (0, 0)),
          pl.BlockSpec((4, 1), lambda i: (0, 0)),
          pl.BlockSpec((1, 1), lambda i: (0, 0)),
      ],
      out_specs=[
          pl.BlockSpec((blk, 1), lambda i: (i, 0)),
          pl.BlockSpec((blk, 8), lambda i: (i, 0)),
      ],
      out_shape=[
          jax.ShapeDtypeStruct((n, 1), jnp.float32),
          jax.ShapeDtypeStruct((n, 8), jnp.float32),
      ],
  )(agg0, agg1, r, x1c, b_eff, w1t, b1r, w4t, b4r, w2et, w2x, b2r, w3t, b3r)
  return (out, emb)


# speedup vs baseline: 1.5067x; 1.5067x over previous
import functools
import jax
import jax.numpy as jnp
from jax import lax
from jax.experimental import pallas as pl
from jax.experimental.pallas import tpu as pltpu
from jax.experimental.pallas import tpu_sc as plsc

D_H = 128
CHUNK = 128
NC, NS = 2, 16
NW = NC * NS
LANES = 16

SPLIT_FN = lambda total: (total // 2, total - total // 2)


def _make_sc_segment_sum(n_nodes, e_pad, cpt0, cpt1):
  assert NS * (cpt0 + cpt1) * CHUNK == e_pad
  rows_per_tile = ((n_nodes + NS - 1) // NS + 7) // 8 * 8
  n_pad = rows_per_tile * NS
  half = CHUNK // 2
  full_zero_chunks = rows_per_tile // half
  rem_zero = rows_per_tile % half
  mesh = plsc.VectorSubcoreMesh(core_axis_name="c", subcore_axis_name="s",
                                num_cores=NC, num_subcores=NS)

  bufc = min(40, max(cpt0, cpt1))
  edges_per_tile = bufc * CHUNK

  @functools.partial(
      pl.kernel,
      out_type=[jax.ShapeDtypeStruct((n_pad, D_H), jnp.float32)] * NC,
      mesh=mesh,
      compiler_params=pltpu.CompilerParams(use_tc_tiling_on_sc=False,
                                           needs_layout_passes=False),
      scratch_types=[
          pltpu.VMEM((edges_per_tile,), jnp.int32),
          pltpu.VMEM((2 * bufc, 1, half), jnp.int32),
          pltpu.VMEM((edges_per_tile,), jnp.float32),
          pltpu.VMEM((CHUNK, D_H // 2), jnp.int32),
          pltpu.VMEM((CHUNK, D_H // 2), jnp.int32),
          pltpu.VMEM((half, D_H), jnp.float32),
          pltpu.VMEM((half, D_H), jnp.float32),
          pltpu.VMEM_SHARED((n_pad, D_H), jnp.float32),
          pltpu.SemaphoreType.DMA,
          pltpu.SemaphoreType.DMA,
          pltpu.SemaphoreType.DMA,
          pltpu.SemaphoreType.DMA,
          pltpu.SemaphoreType.DMA,
      ],
  )
  def sc_kernel(y_hbm, src_hbm, dst_hbm, ew_hbm, out0_hbm, out1_hbm,
                src_v, dst_v, ew_v, gbuf_a, gbuf_b, msg0, msg1, agg_sh,
                gs_a, gs_b, ss_0, ss_1, is_sem):
    c = lax.axis_index("c")
    s = lax.axis_index("s")
    row0 = s * rows_per_tile

    zero16 = jnp.zeros((LANES,), jnp.float32)

    def zrow(i, carry):
      for f in range(D_H // LANES):
        msg0[i, pl.ds(f * LANES, LANES)] = zero16
      return carry

    lax.fori_loop(0, half, zrow, 0)
    for k in range(full_zero_chunks):
      pltpu.sync_copy(msg0, agg_sh.at[pl.ds(row0 + k * half, half)])
    if rem_zero:
      pltpu.sync_copy(msg0.at[pl.ds(0, rem_zero)],
                      agg_sh.at[pl.ds(row0 + full_zero_chunks * half, rem_zero)])

    def issue_gather(ci, buf, gsem):
      return pltpu.async_copy(
          y_hbm.at[src_v.at[pl.ds(ci * CHUNK, CHUNK)]], buf, gsem)

    def issue_scatter(hi, msg, ssem):
      return pltpu.async_copy(msg, agg_sh.at[dst_v.at[hi, 0]], ssem, add=True)

    def wait_scatter(ssem):
      pltpu.make_async_copy(msg0, agg_sh.at[dst_v.at[0, 0]], ssem).wait()

    def wait_gather(buf, gsem):
      pltpu.make_async_copy(y_hbm.at[src_v.at[pl.ds(0, CHUNK)]], buf,
                            gsem).wait()

    mask_hi = jnp.int32(-65536)

    def convert_scale(ci, h, gbuf, msg):
      def grp(g, carry):
        wvec = ew_v[pl.ds(ci * CHUNK + h * half + g * LANES, LANES)]
        for j in range(LANES):
          wv = jnp.full((LANES,), wvec[j], jnp.float32)
          e = h * half + g * LANES + j
          m = g * LANES + j
          for f in range(D_H // 32):
            v = gbuf[e, pl.ds(f * LANES, LANES)]
            lo = plsc.bitcast(lax.shift_left(v, 16), jnp.float32)
            hi = plsc.bitcast(lax.bitwise_and(v, mask_hi), jnp.float32)
            msg[m, pl.ds(f * 32, LANES)] = lo * wv
            msg[m, pl.ds(f * 32 + LANES, LANES)] = hi * wv
        return carry

      lax.fori_loop(0, half // LANES, grp, 0)

    def run(nchunks, tile_chunk0):
      assert nchunks % 2 == 0
      npairs = nchunks // 2
      nedges = nchunks * CHUNK
      ebase = tile_chunk0 * CHUNK
      ia = pltpu.async_copy(src_hbm.at[pl.ds(ebase, nedges)],
                            src_v.at[pl.ds(0, nedges)], is_sem)
      ib = pltpu.async_copy(dst_hbm.at[pl.ds(2 * tile_chunk0, 2 * nchunks)],
                            dst_v.at[pl.ds(0, 2 * nchunks)], is_sem)
      ic = pltpu.async_copy(ew_hbm.at[pl.ds(ebase, nedges)],
                            ew_v.at[pl.ds(0, nedges)], is_sem)
      ia.wait()
      ib.wait()
      ic.wait()
      plsc.subcore_barrier()

      issue_gather(0, gbuf_a, gs_a)
      issue_gather(1, gbuf_b, gs_b)

      def pair_body(p, carry):
        ci = 2 * p
        wait_gather(gbuf_a, gs_a)

        @pl.when(p > 0)
        def _():
          wait_scatter(ss_0)
          wait_scatter(ss_1)

        convert_scale(ci, 0, gbuf_a, msg0)
        issue_scatter(2 * ci, msg0, ss_0)
        convert_scale(ci, 1, gbuf_a, msg1)
        issue_scatter(2 * ci + 1, msg1, ss_1)

        @pl.when(p < npairs - 1)
        def _():
          issue_gather(ci + 2, gbuf_a, gs_a)

        wait_gather(gbuf_b, gs_b)
        wait_scatter(ss_0)
        convert_scale(ci + 1, 0, gbuf_b, msg0)
        issue_scatter(2 * ci + 2, msg0, ss_0)
        wait_scatter(ss_1)
        convert_scale(ci + 1, 1, gbuf_b, msg1)
        issue_scatter(2 * ci + 3, msg1, ss_1)

        @pl.when(p < npairs - 1)
        def _():
          issue_gather(ci + 3, gbuf_b, gs_b)

        return carry

      lax.fori_loop(0, npairs, pair_body, 0)
      wait_scatter(ss_0)
      wait_scatter(ss_1)

    def run_core(cpt, core_base):
      done = 0
      while done < cpt:
        step = min(bufc, cpt - done)
        run(step, core_base + s * cpt + done)
        done += step

    if cpt0 == cpt1:
      run_core(cpt0, c * NS * cpt0)
    else:
      if cpt0:
        @pl.when(c == 0)
        def _():
          run_core(cpt0, 0)
      if cpt1:
        @pl.when(c == 1)
        def _():
          run_core(cpt1, NS * cpt0)
    plsc.subcore_barrier()

    @pl.when(c == 0)
    def _():
      pltpu.sync_copy(agg_sh.at[pl.ds(row0, rows_per_tile)],
                      out0_hbm.at[pl.ds(row0, rows_per_tile)])

    @pl.when(c == 1)
    def _():
      pltpu.sync_copy(agg_sh.at[pl.ds(row0, rows_per_tile)],
                      out1_hbm.at[pl.ds(row0, rows_per_tile)])

  return sc_kernel


def _mm_body(x_ref, w_ref, y_ref, r_ref):
  acc = jnp.dot(x_ref[...], w_ref[...], preferred_element_type=jnp.float32)
  y_ref[...] = acc[:, :D_H].astype(jnp.bfloat16)
  r_ref[...] = acc[:, D_H:]


def _head_body(agg0, agg1, r, x1, brel, w1t, b1, w4t, b4, w2et, w2x, b2, w3t, b3,
               out_ref, emb_ref):
  h = jnp.maximum(agg0[...] + agg1[...] + r[...] + brel[...], 0.0)
  h1 = jnp.maximum(
      jnp.dot(h, w1t[...], preferred_element_type=jnp.float32) + b1[...], 0.0)
  emb = jax.nn.sigmoid(
      jnp.dot(h1, w4t[...], preferred_element_type=jnp.float32) + b4[...])
  emb_ref[...] = emb
  h2 = jax.nn.sigmoid(
      jnp.dot(emb, w2et[...], preferred_element_type=jnp.float32)
      + x1[...] * w2x[...] + b2[...])
  out_ref[...] = jnp.maximum(
      jnp.dot(h2, w3t[...], preferred_element_type=jnp.float32) + b3[...], 0.0)


def kernel(x, x1, edge_weight, W_rel, b_rel, W_root, W1, b1, W4, b4, W2, b2, W3,
           b3, edge_index, loc_num):
  n, d = x.shape
  e = edge_index.shape[1]
  blk = 1000
  nblk = n // blk

  q = jnp.arange(D_H)
  perm = (q // 32) * 32 + (q % 32) // 2 + 16 * ((q % 32) % 2)
  wcat = jnp.concatenate([W_rel[perm], W_root], axis=0).T
  y, r = pl.pallas_call(
      _mm_body,
      grid=(nblk,),
      in_specs=[
          pl.BlockSpec((blk, d), lambda i: (i, 0)),
          pl.BlockSpec((d, 2 * D_H), lambda i: (0, 0)),
      ],
      out_specs=[
          pl.BlockSpec((blk, D_H), lambda i: (i, 0)),
          pl.BlockSpec((blk, D_H), lambda i: (i, 0)),
      ],
      out_shape=[
          jax.ShapeDtypeStruct((n, D_H), jnp.bfloat16),
          jax.ShapeDtypeStruct((n, D_H), jnp.float32),
      ],
  )(x, wcat)
  y_pack = jax.lax.bitcast_convert_type(
      y.reshape(n, D_H // 2, 2), jnp.int32)

  total_cpt = ((e + NS * CHUNK - 1) // (NS * CHUNK) + 1) // 2 * 2
  cpt0, cpt1 = SPLIT_FN(total_cpt)
  e_pad = NS * (cpt0 + cpt1) * CHUNK
  pad = e_pad - e
  src = edge_index[0]
  dst = edge_index[1]
  ew = edge_weight
  if pad:
    zi = jnp.zeros((pad,), jnp.int32)
    src = jnp.concatenate([src, zi])
    dst = jnp.concatenate([dst, zi])
    ew = jnp.concatenate([ew, jnp.zeros((pad,), jnp.float32)])
  dst3 = dst.reshape(e_pad // (CHUNK // 2), 1, CHUNK // 2)
  agg0, agg1 = _make_sc_segment_sum(n, e_pad, cpt0, cpt1)(y_pack, src, dst3, ew)

  delta = (jnp.asarray(loc_num) - n).astype(jnp.float32)
  b_eff = (b_rel + delta).reshape(1, D_H)
  x1c = x1.reshape(n, 1)
  w1t = W1.T
  b1r = b1.reshape(1, 32)
  w4t = W4.T
  b4r = b4.reshape(1, 8)
  w2et = W2[:, :8].T
  w2x = W2[:, 8].reshape(1, 4)
  b2r = b2.reshape(1, 4)
  w3t = W3.T
  b3r = b3.reshape(1, 1)

  out, emb = pl.pallas_call(
      _head_body,
      grid=(nblk,),
      in_specs=[
          pl.BlockSpec((blk, D_H), lambda i: (i, 0)),
          pl.BlockSpec((blk, D_H), lambda i: (i, 0)),
          pl.BlockSpec((blk, D_H), lambda i: (i, 0)),
          pl.BlockSpec((blk, 1), lambda i: (i, 0)),
          pl.BlockSpec((1, D_H), lambda i: (0, 0)),
          pl.BlockSpec((D_H, 32), lambda i: (0, 0)),
          pl.BlockSpec((1, 32), lambda i: (0, 0)),
          pl.BlockSpec((32, 8), lambda i: (0, 0)),
          pl.BlockSpec((1, 8), lambda i: (0, 0)),
          pl.BlockSpec((8, 4), lambda i: (0, 0)),
          pl.BlockSpec((1, 4), lambda i: (0, 0)),
          pl.BlockSpec((1, 4), lambda i: (0, 0)),
          pl.BlockSpec((4, 1), lambda i: (0, 0)),
          pl.BlockSpec((1, 1), lambda i: (0, 0)),
      ],
      out_specs=[
          pl.BlockSpec((blk, 1), lambda i: (i, 0)),
          pl.BlockSpec((blk, 8), lambda i: (i, 0)),
      ],
      out_shape=[
          jax.ShapeDtypeStruct((n, 1), jnp.float32),
          jax.ShapeDtypeStruct((n, 8), jnp.float32),
      ],
  )(agg0, agg1, r, x1c, b_eff, w1t, b1r, w4t, b4r, w2et, w2x, b2r, w3t, b3r)
  return (out, emb)


# speedup vs baseline: 1.5783x; 1.0475x over previous
import functools
import jax
import jax.numpy as jnp
from jax import lax
from jax.experimental import pallas as pl
from jax.experimental.pallas import tpu as pltpu
from jax.experimental.pallas import tpu_sc as plsc

D_H = 128
CHUNK = 128
NC, NS = 2, 16
NW = NC * NS
LANES = 16

SPLIT_FN = lambda total: (total // 2, total - total // 2)


def _make_sc_segment_sum(n_nodes, e_pad, cpt0, cpt1):
  assert NS * (cpt0 + cpt1) * CHUNK == e_pad
  rows_per_tile = ((n_nodes + NS - 1) // NS + 7) // 8 * 8
  n_pad = rows_per_tile * NS
  half = CHUNK // 2
  full_zero_chunks = rows_per_tile // half
  rem_zero = rows_per_tile % half
  mesh = plsc.VectorSubcoreMesh(core_axis_name="c", subcore_axis_name="s",
                                num_cores=NC, num_subcores=NS)

  bufc = min(40, max(cpt0, cpt1))
  edges_per_tile = bufc * CHUNK

  @functools.partial(
      pl.kernel,
      out_type=[jax.ShapeDtypeStruct((n_pad, D_H), jnp.float32)] * NC,
      mesh=mesh,
      compiler_params=pltpu.CompilerParams(use_tc_tiling_on_sc=False,
                                           needs_layout_passes=False),
      scratch_types=[
          pltpu.VMEM((edges_per_tile,), jnp.int32),
          pltpu.VMEM((2 * bufc, 1, half), jnp.int32),
          pltpu.VMEM((edges_per_tile,), jnp.float32),
          pltpu.VMEM((CHUNK, D_H // 2), jnp.int32),
          pltpu.VMEM((CHUNK, D_H // 2), jnp.int32),
          pltpu.VMEM((half, D_H), jnp.float32),
          pltpu.VMEM((half, D_H), jnp.float32),
          pltpu.VMEM_SHARED((n_pad, D_H), jnp.float32),
          pltpu.SemaphoreType.DMA,
          pltpu.SemaphoreType.DMA,
          pltpu.SemaphoreType.DMA,
          pltpu.SemaphoreType.DMA,
          pltpu.SemaphoreType.DMA,
      ],
  )
  def sc_kernel(y_hbm, src_hbm, dst_hbm, ew_hbm, out0_hbm, out1_hbm,
                src_v, dst_v, ew_v, gbuf_a, gbuf_b, msg0, msg1, agg_sh,
                gs_a, gs_b, ss_0, ss_1, is_sem):
    c = lax.axis_index("c")
    s = lax.axis_index("s")
    row0 = s * rows_per_tile

    zero16 = jnp.zeros((LANES,), jnp.float32)

    def zrow(i, carry):
      for f in range(D_H // LANES):
        msg0[i, pl.ds(f * LANES, LANES)] = zero16
      return carry

    lax.fori_loop(0, half, zrow, 0)
    for k in range(full_zero_chunks):
      pltpu.sync_copy(msg0, agg_sh.at[pl.ds(row0 + k * half, half)])
    if rem_zero:
      pltpu.sync_copy(msg0.at[pl.ds(0, rem_zero)],
                      agg_sh.at[pl.ds(row0 + full_zero_chunks * half, rem_zero)])

    def issue_gather(ci, buf, gsem):
      return pltpu.async_copy(
          y_hbm.at[src_v.at[pl.ds(ci * CHUNK, CHUNK)]], buf, gsem)

    def issue_scatter(hi, msg, ssem):
      return pltpu.async_copy(msg, agg_sh.at[dst_v.at[hi, 0]], ssem, add=True)

    def wait_scatter(ssem):
      pltpu.make_async_copy(msg0, agg_sh.at[dst_v.at[0, 0]], ssem).wait()

    def wait_gather(buf, gsem):
      pltpu.make_async_copy(y_hbm.at[src_v.at[pl.ds(0, CHUNK)]], buf,
                            gsem).wait()

    mask_hi = jnp.int32(-65536)

    def convert_scale(ci, h, gbuf, msg):
      def grp(g, carry):
        wvec = ew_v[pl.ds(ci * CHUNK + h * half + g * LANES, LANES)]
        for j in range(LANES):
          wv = jnp.full((LANES,), wvec[j], jnp.float32)
          e = h * half + g * LANES + j
          m = g * LANES + j
          for f in range(D_H // 32):
            v = gbuf[e, pl.ds(f * LANES, LANES)]
            lo = plsc.bitcast(lax.shift_left(v, 16), jnp.float32)
            hi = plsc.bitcast(lax.bitwise_and(v, mask_hi), jnp.float32)
            msg[m, pl.ds(f * 32, LANES)] = lo * wv
            msg[m, pl.ds(f * 32 + LANES, LANES)] = hi * wv
        return carry

      lax.fori_loop(0, half // LANES, grp, 0)

    def run(nchunks, tile_chunk0):
      assert nchunks % 2 == 0
      npairs = nchunks // 2
      nedges = nchunks * CHUNK
      ebase = tile_chunk0 * CHUNK
      ia = pltpu.async_copy(src_hbm.at[pl.ds(ebase, nedges)],
                            src_v.at[pl.ds(0, nedges)], is_sem)
      ib = pltpu.async_copy(dst_hbm.at[pl.ds(2 * tile_chunk0, 2 * nchunks)],
                            dst_v.at[pl.ds(0, 2 * nchunks)], is_sem)
      ic = pltpu.async_copy(ew_hbm.at[pl.ds(ebase, nedges)],
                            ew_v.at[pl.ds(0, nedges)], is_sem)
      ia.wait()
      ib.wait()
      ic.wait()
      plsc.subcore_barrier()

      issue_gather(0, gbuf_a, gs_a)
      issue_gather(1, gbuf_b, gs_b)

      def pair_body(p, carry):
        ci = 2 * p
        wait_gather(gbuf_a, gs_a)

        @pl.when(p > 0)
        def _():
          wait_scatter(ss_0)
          wait_scatter(ss_1)

        convert_scale(ci, 0, gbuf_a, msg0)
        issue_scatter(2 * ci, msg0, ss_0)
        convert_scale(ci, 1, gbuf_a, msg1)
        issue_scatter(2 * ci + 1, msg1, ss_1)

        @pl.when(p < npairs - 1)
        def _():
          issue_gather(ci + 2, gbuf_a, gs_a)

        wait_gather(gbuf_b, gs_b)
        wait_scatter(ss_0)
        convert_scale(ci + 1, 0, gbuf_b, msg0)
        issue_scatter(2 * ci + 2, msg0, ss_0)
        wait_scatter(ss_1)
        convert_scale(ci + 1, 1, gbuf_b, msg1)
        issue_scatter(2 * ci + 3, msg1, ss_1)

        @pl.when(p < npairs - 1)
        def _():
          issue_gather(ci + 3, gbuf_b, gs_b)

        return carry

      lax.fori_loop(0, npairs, pair_body, 0)
      wait_scatter(ss_0)
      wait_scatter(ss_1)

    def run_core(cpt, core_base):
      done = 0
      while done < cpt:
        step = min(bufc, cpt - done)
        run(step, core_base + s * cpt + done)
        done += step

    if cpt0 == cpt1:
      run_core(cpt0, c * NS * cpt0)
    else:
      if cpt0:
        @pl.when(c == 0)
        def _():
          run_core(cpt0, 0)
      if cpt1:
        @pl.when(c == 1)
        def _():
          run_core(cpt1, NS * cpt0)
    plsc.subcore_barrier()

    @pl.when(c == 0)
    def _():
      pltpu.sync_copy(agg_sh.at[pl.ds(row0, rows_per_tile)],
                      out0_hbm.at[pl.ds(row0, rows_per_tile)])

    @pl.when(c == 1)
    def _():
      pltpu.sync_copy(agg_sh.at[pl.ds(row0, rows_per_tile)],
                      out1_hbm.at[pl.ds(row0, rows_per_tile)])

  return sc_kernel


def _mm_body(x_ref, w_ref, y_ref, r_ref):
  acc = jnp.dot(x_ref[...], w_ref[...], preferred_element_type=jnp.float32)
  lo = jax.lax.bitcast_convert_type(acc[:, :D_H // 2], jnp.int32)
  hi = jax.lax.bitcast_convert_type(acc[:, D_H // 2:D_H], jnp.int32)
  rnd = jnp.int32(0x7FFF)
  one = jnp.int32(1)
  lo16 = jax.lax.shift_right_logical(
      lo + rnd + (jax.lax.shift_right_logical(lo, 16) & one), 16)
  hi16 = (hi + rnd + (jax.lax.shift_right_logical(hi, 16) & one)) & jnp.int32(
      -65536)
  y_ref[...] = lo16 | hi16
  r_ref[...] = acc[:, D_H:]


def _head_body(agg0, agg1, r, x1, brel, w1t, b1, w4t, b4, w2et, w2x, b2, w3t, b3,
               out_ref, emb_ref):
  h = jnp.maximum(agg0[...] + agg1[...] + r[...] + brel[...], 0.0)
  h1 = jnp.maximum(
      jnp.dot(h, w1t[...], preferred_element_type=jnp.float32) + b1[...], 0.0)
  emb = jax.nn.sigmoid(
      jnp.dot(h1, w4t[...], preferred_element_type=jnp.float32) + b4[...])
  emb_ref[...] = emb
  h2 = jax.nn.sigmoid(
      jnp.dot(emb, w2et[...], preferred_element_type=jnp.float32)
      + x1[...] * w2x[...] + b2[...])
  out_ref[...] = jnp.maximum(
      jnp.dot(h2, w3t[...], preferred_element_type=jnp.float32) + b3[...], 0.0)


def kernel(x, x1, edge_weight, W_rel, b_rel, W_root, W1, b1, W4, b4, W2, b2, W3,
           b3, edge_index, loc_num):
  n, d = x.shape
  e = edge_index.shape[1]
  blk = 1000
  nblk = n // blk

  q = jnp.arange(D_H)
  h64 = D_H // 2
  perm = jnp.where(q < h64, 32 * (q // 16) + q % 16,
                   32 * ((q - h64) // 16) + 16 + (q - h64) % 16)
  wcat = jnp.concatenate([W_rel[perm], W_root], axis=0).T
  y, r = pl.pallas_call(
      _mm_body,
      grid=(nblk,),
      in_specs=[
          pl.BlockSpec((blk, d), lambda i: (i, 0)),
          pl.BlockSpec((d, 2 * D_H), lambda i: (0, 0)),
      ],
      out_specs=[
          pl.BlockSpec((blk, D_H // 2), lambda i: (i, 0)),
          pl.BlockSpec((blk, D_H), lambda i: (i, 0)),
      ],
      out_shape=[
          jax.ShapeDtypeStruct((n, D_H // 2), jnp.int32),
          jax.ShapeDtypeStruct((n, D_H), jnp.float32),
      ],
  )(x, wcat)
  y_pack = y

  total_cpt = ((e + NS * CHUNK - 1) // (NS * CHUNK) + 1) // 2 * 2
  cpt0, cpt1 = SPLIT_FN(total_cpt)
  e_pad = NS * (cpt0 + cpt1) * CHUNK
  pad = e_pad - e
  src = edge_index[0]
  dst = edge_index[1]
  ew = edge_weight
  if pad:
    zi = jnp.zeros((pad,), jnp.int32)
    src = jnp.concatenate([src, zi])
    dst = jnp.concatenate([dst, zi])
    ew = jnp.concatenate([ew, jnp.zeros((pad,), jnp.float32)])
  dst3 = dst.reshape(e_pad // (CHUNK // 2), 1, CHUNK // 2)
  agg0, agg1 = _make_sc_segment_sum(n, e_pad, cpt0, cpt1)(y_pack, src, dst3, ew)

  delta = (jnp.asarray(loc_num) - n).astype(jnp.float32)
  b_eff = (b_rel + delta).reshape(1, D_H)
  x1c = x1.reshape(n, 1)
  w1t = W1.T
  b1r = b1.reshape(1, 32)
  w4t = W4.T
  b4r = b4.reshape(1, 8)
  w2et = W2[:, :8].T
  w2x = W2[:, 8].reshape(1, 4)
  b2r = b2.reshape(1, 4)
  w3t = W3.T
  b3r = b3.reshape(1, 1)

  out, emb = pl.pallas_call(
      _head_body,
      grid=(nblk,),
      in_specs=[
          pl.BlockSpec((blk, D_H), lambda i: (i, 0)),
          pl.BlockSpec((blk, D_H), lambda i: (i, 0)),
          pl.BlockSpec((blk, D_H), lambda i: (i, 0)),
          pl.BlockSpec((blk, 1), lambda i: (i, 0)),
          pl.BlockSpec((1, D_H), lambda i: (0, 0)),
          pl.BlockSpec((D_H, 32), lambda i: (0, 0)),
          pl.BlockSpec((1, 32), lambda i: (0, 0)),
          pl.BlockSpec((32, 8), lambda i: (0, 0)),
          pl.BlockSpec((1, 8), lambda i: (0, 0)),
          pl.BlockSpec((8, 4), lambda i: (0, 0)),
          pl.BlockSpec((1, 4), lambda i: (0, 0)),
          pl.BlockSpec((1, 4), lambda i: (0, 0)),
          pl.BlockSpec((4, 1), lambda i: (0, 0)),
          pl.BlockSpec((1, 1), lambda i: (0, 0)),
      ],
      out_specs=[
          pl.BlockSpec((blk, 1), lambda i: (i, 0)),
          pl.BlockSpec((blk, 8), lambda i: (i, 0)),
      ],
      out_shape=[
          jax.ShapeDtypeStruct((n, 1), jnp.float32),
          jax.ShapeDtypeStruct((n, 8), jnp.float32),
      ],
  )(agg0, agg1, r, x1c, b_eff, w1t, b1r, w4t, b4r, w2et, w2x, b2r, w3t, b3r)
  return (out, emb)


# speedup vs baseline: 1.6452x; 1.0423x over previous
import functools
import jax
import jax.numpy as jnp
from jax import lax
from jax.experimental import pallas as pl
from jax.experimental.pallas import tpu as pltpu
from jax.experimental.pallas import tpu_sc as plsc

D_H = 128
CHUNK = 128
NC, NS = 2, 16
NW = NC * NS
LANES = 16


def _make_sc_segment_sum(n_nodes, e_pad):
  assert e_pad % (NS * CHUNK) == 0
  chunks_per_tile = e_pad // (NS * CHUNK)
  assert chunks_per_tile % 2 == 0
  npairs = chunks_per_tile // 2
  rows_per_tile = ((n_nodes + NS - 1) // NS + 7) // 8 * 8
  n_pad = rows_per_tile * NS
  last_rows = n_nodes - (NS - 1) * rows_per_tile
  assert 0 < last_rows <= rows_per_tile and last_rows % 8 == 0
  half = CHUNK // 2
  DF = D_H // 2
  WPN = DF // 2
  full_zero_chunks = rows_per_tile // half
  rem_zero = rows_per_tile % half
  edges_per_tile = chunks_per_tile * CHUNK
  mesh = plsc.VectorSubcoreMesh(core_axis_name="c", subcore_axis_name="s",
                                num_cores=NC, num_subcores=NS)

  @functools.partial(
      pl.kernel,
      out_type=[jax.ShapeDtypeStruct((n_pad, DF), jnp.float32)] * NC,
      mesh=mesh,
      compiler_params=pltpu.CompilerParams(use_tc_tiling_on_sc=False,
                                           needs_layout_passes=False),
      scratch_types=[
          pltpu.VMEM((edges_per_tile,), jnp.int32),
          pltpu.VMEM((2 * chunks_per_tile, 1, half), jnp.int32),
          pltpu.VMEM((edges_per_tile,), jnp.float32),
          pltpu.VMEM((CHUNK, WPN), jnp.int32),
          pltpu.VMEM((CHUNK, WPN), jnp.int32),
          pltpu.VMEM((half, DF), jnp.float32),
          pltpu.VMEM((half, DF), jnp.float32),
          pltpu.VMEM_SHARED((n_pad, WPN), jnp.int32),
          pltpu.VMEM_SHARED((n_pad, DF), jnp.float32),
          pltpu.SemaphoreType.DMA,
          pltpu.SemaphoreType.DMA,
          pltpu.SemaphoreType.DMA,
          pltpu.SemaphoreType.DMA,
          pltpu.SemaphoreType.DMA,
      ],
  )
  def sc_kernel(y_hbm, src_hbm, dst_hbm, ew_hbm, out0_hbm, out1_hbm,
                src_v, dst_v, ew_v, gbuf_a, gbuf_b, msg0, msg1,
                table_sh, agg_sh, gs_a, gs_b, ss_0, ss_1, is_sem):
    c = lax.axis_index("c")
    s = lax.axis_index("s")
    row0 = s * rows_per_tile
    ebase = s * edges_per_tile
    tile_chunk0 = s * chunks_per_tile

    ia = pltpu.async_copy(src_hbm.at[pl.ds(ebase, edges_per_tile)], src_v,
                          is_sem)
    ib = pltpu.async_copy(
        dst_hbm.at[pl.ds(2 * tile_chunk0, 2 * chunks_per_tile)], dst_v, is_sem)
    ic = pltpu.async_copy(ew_hbm.at[pl.ds(ebase, edges_per_tile)], ew_v,
                          is_sem)

    def load_table(nrows):
      @pl.when(c == 0)
      def _():
        pltpu.sync_copy(y_hbm.at[pl.ds(row0, nrows), pl.ds(0, WPN)],
                        table_sh.at[pl.ds(row0, nrows)])

      @pl.when(c == 1)
      def _():
        pltpu.sync_copy(y_hbm.at[pl.ds(row0, nrows), pl.ds(WPN, WPN)],
                        table_sh.at[pl.ds(row0, nrows)])

    @pl.when(s < NS - 1)
    def _():
      load_table(rows_per_tile)

    @pl.when(s == NS - 1)
    def _():
      load_table(last_rows)

    zero16 = jnp.zeros((LANES,), jnp.float32)

    def zrow(i, carry):
      for f in range(DF // LANES):
        msg0[i, pl.ds(f * LANES, LANES)] = zero16
      return carry

    lax.fori_loop(0, half, zrow, 0)
    for k in range(full_zero_chunks):
      pltpu.sync_copy(msg0, agg_sh.at[pl.ds(row0 + k * half, half)])
    if rem_zero:
      pltpu.sync_copy(msg0.at[pl.ds(0, rem_zero)],
                      agg_sh.at[pl.ds(row0 + full_zero_chunks * half, rem_zero)])
    ia.wait()
    ib.wait()
    ic.wait()
    plsc.subcore_barrier()

    def issue_gather(ci, buf, gsem):
      return pltpu.async_copy(
          table_sh.at[src_v.at[pl.ds(ci * CHUNK, CHUNK)]], buf, gsem)

    def issue_scatter(hi, msg, ssem):
      return pltpu.async_copy(msg, agg_sh.at[dst_v.at[hi, 0]], ssem, add=True)

    def wait_scatter(ssem):
      pltpu.make_async_copy(msg0, agg_sh.at[dst_v.at[0, 0]], ssem).wait()

    def wait_gather(buf, gsem):
      pltpu.make_async_copy(table_sh.at[src_v.at[pl.ds(0, CHUNK)]], buf,
                            gsem).wait()

    mask_hi = jnp.int32(-65536)

    def convert_scale(ci, h, gbuf, msg):
      def grp(g, carry):
        wvec = ew_v[pl.ds(ci * CHUNK + h * half + g * LANES, LANES)]
        for j in range(LANES):
          wv = jnp.full((LANES,), wvec[j], jnp.float32)
          e = h * half + g * LANES + j
          m = g * LANES + j
          for f in range(WPN // LANES):
            v = gbuf[e, pl.ds(f * LANES, LANES)]
            lo = plsc.bitcast(lax.shift_left(v, 16), jnp.float32)
            hi = plsc.bitcast(lax.bitwise_and(v, mask_hi), jnp.float32)
            msg[m, pl.ds(f * 32, LANES)] = lo * wv
            msg[m, pl.ds(f * 32 + LANES, LANES)] = hi * wv
        return carry

      lax.fori_loop(0, half // LANES, grp, 0)

    issue_gather(0, gbuf_a, gs_a)
    issue_gather(1, gbuf_b, gs_b)

    def pair_body(p, carry):
      ci = 2 * p
      wait_gather(gbuf_a, gs_a)

      @pl.when(p > 0)
      def _():
        wait_scatter(ss_0)
        wait_scatter(ss_1)

      convert_scale(ci, 0, gbuf_a, msg0)
      issue_scatter(2 * ci, msg0, ss_0)
      convert_scale(ci, 1, gbuf_a, msg1)
      issue_scatter(2 * ci + 1, msg1, ss_1)

      @pl.when(p < npairs - 1)
      def _():
        issue_gather(ci + 2, gbuf_a, gs_a)

      wait_gather(gbuf_b, gs_b)
      wait_scatter(ss_0)
      convert_scale(ci + 1, 0, gbuf_b, msg0)
      issue_scatter(2 * ci + 2, msg0, ss_0)
      wait_scatter(ss_1)
      convert_scale(ci + 1, 1, gbuf_b, msg1)
      issue_scatter(2 * ci + 3, msg1, ss_1)

      @pl.when(p < npairs - 1)
      def _():
        issue_gather(ci + 3, gbuf_b, gs_b)

      return carry

    lax.fori_loop(0, npairs, pair_body, 0)
    wait_scatter(ss_0)
    wait_scatter(ss_1)
    plsc.subcore_barrier()

    @pl.when(c == 0)
    def _():
      pltpu.sync_copy(agg_sh.at[pl.ds(row0, rows_per_tile)],
                      out0_hbm.at[pl.ds(row0, rows_per_tile)])

    @pl.when(c == 1)
    def _():
      pltpu.sync_copy(agg_sh.at[pl.ds(row0, rows_per_tile)],
                      out1_hbm.at[pl.ds(row0, rows_per_tile)])

  return sc_kernel


def _mm_body(x_ref, w_ref, y_ref, r_ref):
  acc = jnp.dot(x_ref[...], w_ref[...], preferred_element_type=jnp.float32)
  lo = jax.lax.bitcast_convert_type(acc[:, :D_H // 2], jnp.int32)
  hi = jax.lax.bitcast_convert_type(acc[:, D_H // 2:D_H], jnp.int32)
  rnd = jnp.int32(0x7FFF)
  one = jnp.int32(1)
  lo16 = jax.lax.shift_right_logical(
      lo + rnd + (jax.lax.shift_right_logical(lo, 16) & one), 16)
  hi16 = (hi + rnd + (jax.lax.shift_right_logical(hi, 16) & one)) & jnp.int32(
      -65536)
  y_ref[...] = lo16 | hi16
  r_ref[...] = acc[:, D_H:]


def _head_body(agg0, agg1, r, x1, brel, w1t, b1, w4t, b4, w2et, w2x, b2, w3t, b3,
               out_ref, emb_ref):
  agg = jnp.concatenate([agg0[...], agg1[...]], axis=1)
  h = jnp.maximum(agg + r[...] + brel[...], 0.0)
  h1 = jnp.maximum(
      jnp.dot(h, w1t[...], preferred_element_type=jnp.float32) + b1[...], 0.0)
  emb = jax.nn.sigmoid(
      jnp.dot(h1, w4t[...], preferred_element_type=jnp.float32) + b4[...])
  emb_ref[...] = emb
  h2 = jax.nn.sigmoid(
      jnp.dot(emb, w2et[...], preferred_element_type=jnp.float32)
      + x1[...] * w2x[...] + b2[...])
  out_ref[...] = jnp.maximum(
      jnp.dot(h2, w3t[...], preferred_element_type=jnp.float32) + b3[...], 0.0)


def kernel(x, x1, edge_weight, W_rel, b_rel, W_root, W1, b1, W4, b4, W2, b2, W3,
           b3, edge_index, loc_num):
  n, d = x.shape
  e = edge_index.shape[1]
  blk = 1000
  nblk = n // blk

  q = jnp.arange(D_H)
  h64 = D_H // 2
  w = jnp.where(q < h64, q, q - h64)
  k = w % 32
  base = 64 * (w // 32) + 32 * (k // 16) + k % 16
  perm = jnp.where(q < h64, base, base + 16)
  wcat = jnp.concatenate([W_rel[perm], W_root], axis=0).T
  y, r = pl.pallas_call(
      _mm_body,
      grid=(nblk,),
      in_specs=[
          pl.BlockSpec((blk, d), lambda i: (i, 0)),
          pl.BlockSpec((d, 2 * D_H), lambda i: (0, 0)),
      ],
      out_specs=[
          pl.BlockSpec((blk, D_H // 2), lambda i: (i, 0)),
          pl.BlockSpec((blk, D_H), lambda i: (i, 0)),
      ],
      out_shape=[
          jax.ShapeDtypeStruct((n, D_H // 2), jnp.int32),
          jax.ShapeDtypeStruct((n, D_H), jnp.float32),
      ],
  )(x, wcat)
  y_pack = y

  cpt = ((e + NS * CHUNK - 1) // (NS * CHUNK) + 1) // 2 * 2
  e_pad = NS * cpt * CHUNK
  pad = e_pad - e
  src = edge_index[0]
  dst = edge_index[1]
  ew = edge_weight
  if pad:
    zi = jnp.zeros((pad,), jnp.int32)
    src = jnp.concatenate([src, zi])
    dst = jnp.concatenate([dst, zi])
    ew = jnp.concatenate([ew, jnp.zeros((pad,), jnp.float32)])
  dst3 = dst.reshape(e_pad // (CHUNK // 2), 1, CHUNK // 2)
  agg0, agg1 = _make_sc_segment_sum(n, e_pad)(y_pack, src, dst3, ew)

  delta = (jnp.asarray(loc_num) - n).astype(jnp.float32)
  b_eff = (b_rel + delta).reshape(1, D_H)
  x1c = x1.reshape(n, 1)
  w1t = W1.T
  b1r = b1.reshape(1, 32)
  w4t = W4.T
  b4r = b4.reshape(1, 8)
  w2et = W2[:, :8].T
  w2x = W2[:, 8].reshape(1, 4)
  b2r = b2.reshape(1, 4)
  w3t = W3.T
  b3r = b3.reshape(1, 1)

  out, emb = pl.pallas_call(
      _head_body,
      grid=(nblk,),
      in_specs=[
          pl.BlockSpec((blk, D_H // 2), lambda i: (i, 0)),
          pl.BlockSpec((blk, D_H // 2), lambda i: (i, 0)),
          pl.BlockSpec((blk, D_H), lambda i: (i, 0)),
          pl.BlockSpec((blk, 1), lambda i: (i, 0)),
          pl.BlockSpec((1, D_H), lambda i: (0, 0)),
          pl.BlockSpec((D_H, 32), lambda i: (0, 0)),
          pl.BlockSpec((1, 32), lambda i: (0, 0)),
          pl.BlockSpec((32, 8), lambda i: (0, 0)),
          pl.BlockSpec((1, 8), lambda i: (0, 0)),
          pl.BlockSpec((8, 4), lambda i: (0, 0)),
          pl.BlockSpec((1, 4), lambda i: (0, 0)),
          pl.BlockSpec((1, 4), lambda i: (0, 0)),
          pl.BlockSpec((4, 1), lambda i: (0, 0)),
          pl.BlockSpec((1, 1), lambda i: (0, 0)),
      ],
      out_specs=[
          pl.BlockSpec((blk, 1), lambda i: (i, 0)),
          pl.BlockSpec((blk, 8), lambda i: (i, 0)),
      ],
      out_shape=[
          jax.ShapeDtypeStruct((n, 1), jnp.float32),
          jax.ShapeDtypeStruct((n, 8), jnp.float32),
      ],
  )(agg0, agg1, r, x1c, b_eff, w1t, b1r, w4t, b4r, w2et, w2x, b2r, w3t, b3r)
  return (out, emb)
